# baseline (speedup 1.0000x reference)
import functools

import jax
import jax.numpy as jnp
import numpy as np
from jax.experimental import pallas as pl
from jax.experimental.pallas import tpu as pltpu

_LANE = 1024
_BLOCK_ROWS = 256


def _intervals_for(L, ratio=0.15, seed=0):
    rng = np.random.default_rng(seed)
    min_win, max_win = 0, int(0.05 * L)
    intervals, durations = [], []
    while sum(durations) < ratio * L:
        random_start = int(rng.integers(0, L - max_win))
        random_end = random_start + int(rng.integers(min_win, max_win))
        random_win = np.arange(random_start, random_end)
        intersections = [len(np.intersect1d(p, random_win)) for p in intervals]
        if sum(intersections) >= random_end - random_start:
            continue
        intervals.append(random_win)
        durations.append(random_end - random_start - sum(intersections))
    return intervals


@functools.lru_cache(maxsize=None)
def _runs_for(L):
    idx = np.arange(L, dtype=np.int64)
    for win in _intervals_for(L):
        src = idx[win[0] - 1] if win[0] else idx[0]
        idx[win] = src
    masked = np.flatnonzero(idx != np.arange(L))
    runs = []
    if masked.size:
        start = prev = int(masked[0])
        val = int(idx[start])
        for i in masked[1:]:
            i = int(i)
            if i == prev + 1 and int(idx[i]) == val:
                prev = i
            else:
                runs.append((start, prev + 1, val))
                start = prev = i
                val = int(idx[i])
        runs.append((start, prev + 1, val))
    return tuple(runs)


def _mask_body(runs, block_elems, fills_ref, x_ref, o_ref):
    pid = pl.program_id(0)
    o_ref[...] = x_ref[...]
    for r, (s, e, _) in enumerate(runs):
        fb, lb = s // block_elems, (e - 1) // block_elems

        @pl.when((pid >= fb) & (pid <= lb))
        def _fill(r=r, s=s, e=e):
            base = pid * block_elems
            shape = o_ref.shape
            pos = (
                base
                + jax.lax.broadcasted_iota(jnp.int32, shape, 0) * _LANE
                + jax.lax.broadcasted_iota(jnp.int32, shape, 1)
            )
            o_ref[...] = jnp.where((pos >= s) & (pos < e), fills_ref[0, r], o_ref[...])


def kernel(x):
    L = x.shape[-1]
    runs = _runs_for(L)
    rows = L // _LANE
    x2 = x.reshape(rows, _LANE)
    block_elems = _BLOCK_ROWS * _LANE
    grid = rows // _BLOCK_ROWS
    srcs = jnp.asarray([src for (_, _, src) in runs], dtype=jnp.int32)
    fills = x[srcs].reshape(1, len(runs))
    out = pl.pallas_call(
        functools.partial(_mask_body, runs, block_elems),
        grid=(grid,),
        in_specs=[
            pl.BlockSpec((1, len(runs)), lambda i: (0, 0)),
            pl.BlockSpec((_BLOCK_ROWS, _LANE), lambda i: (i, 0)),
        ],
        out_specs=pl.BlockSpec((_BLOCK_ROWS, _LANE), lambda i: (i, 0)),
        out_shape=jax.ShapeDtypeStruct((rows, _LANE), x.dtype),
        compiler_params=pltpu.CompilerParams(
            dimension_semantics=("parallel",),
        ),
    )(fills, x2)
    return out.reshape(x.shape)


# speedup vs baseline: 1.0788x; 1.0788x over previous
import functools

import jax
import jax.numpy as jnp
import numpy as np
from jax.experimental import pallas as pl
from jax.experimental.pallas import tpu as pltpu

_LANE = 1024
_BLOCK_ROWS = 512


def _intervals_for(L, ratio=0.15, seed=0):
    rng = np.random.default_rng(seed)
    min_win, max_win = 0, int(0.05 * L)
    intervals, durations = [], []
    while sum(durations) < ratio * L:
        random_start = int(rng.integers(0, L - max_win))
        random_end = random_start + int(rng.integers(min_win, max_win))
        random_win = np.arange(random_start, random_end)
        intersections = [len(np.intersect1d(p, random_win)) for p in intervals]
        if sum(intersections) >= random_end - random_start:
            continue
        intervals.append(random_win)
        durations.append(random_end - random_start - sum(intersections))
    return intervals


@functools.lru_cache(maxsize=None)
def _runs_for(L):
    idx = np.arange(L, dtype=np.int64)
    for win in _intervals_for(L):
        src = idx[win[0] - 1] if win[0] else idx[0]
        idx[win] = src
    masked = np.flatnonzero(idx != np.arange(L))
    runs = []
    if masked.size:
        start = prev = int(masked[0])
        val = int(idx[start])
        for i in masked[1:]:
            i = int(i)
            if i == prev + 1 and int(idx[i]) == val:
                prev = i
            else:
                runs.append((start, prev + 1, val))
                start = prev = i
                val = int(idx[i])
        runs.append((start, prev + 1, val))
    return tuple(runs)


def _mask_body(runs, block_elems, fills_ref, x_ref, o_ref):
    pid = pl.program_id(0)
    o_ref[...] = x_ref[...]
    for r, (s, e, _) in enumerate(runs):
        fb, lb = s // block_elems, (e - 1) // block_elems

        @pl.when((pid >= fb) & (pid <= lb))
        def _fill(r=r, s=s, e=e):
            base = pid * block_elems
            shape = o_ref.shape
            pos = (
                base
                + jax.lax.broadcasted_iota(jnp.int32, shape, 0) * _LANE
                + jax.lax.broadcasted_iota(jnp.int32, shape, 1)
            )
            o_ref[...] = jnp.where((pos >= s) & (pos < e), fills_ref[0, r], o_ref[...])


def kernel(x):
    L = x.shape[-1]
    runs = _runs_for(L)
    rows = L // _LANE
    x2 = x.reshape(rows, _LANE)
    block_elems = _BLOCK_ROWS * _LANE
    grid = rows // _BLOCK_ROWS
    srcs = jnp.asarray([src for (_, _, src) in runs], dtype=jnp.int32)
    fills = x[srcs].reshape(1, len(runs))
    out = pl.pallas_call(
        functools.partial(_mask_body, runs, block_elems),
        grid=(grid,),
        in_specs=[
            pl.BlockSpec((1, len(runs)), lambda i: (0, 0)),
            pl.BlockSpec((_BLOCK_ROWS, _LANE), lambda i: (i, 0)),
        ],
        out_specs=pl.BlockSpec((_BLOCK_ROWS, _LANE), lambda i: (i, 0)),
        out_shape=jax.ShapeDtypeStruct((rows, _LANE), x.dtype),
        compiler_params=pltpu.CompilerParams(
            dimension_semantics=("parallel",),
        ),
    )(fills, x2)
    return out.reshape(x.shape)


# speedup vs baseline: 4.1195x; 3.8186x over previous
import functools

import jax
import jax.numpy as jnp
import numpy as np
from jax.experimental import pallas as pl
from jax.experimental.pallas import tpu as pltpu

_BLOCK = 512 * 1024


def _intervals_for(L, ratio=0.15, seed=0):
    rng = np.random.default_rng(seed)
    min_win, max_win = 0, int(0.05 * L)
    intervals, durations = [], []
    while sum(durations) < ratio * L:
        random_start = int(rng.integers(0, L - max_win))
        random_end = random_start + int(rng.integers(min_win, max_win))
        random_win = np.arange(random_start, random_end)
        intersections = [len(np.intersect1d(p, random_win)) for p in intervals]
        if sum(intersections) >= random_end - random_start:
            continue
        intervals.append(random_win)
        durations.append(random_end - random_start - sum(intersections))
    return intervals


@functools.lru_cache(maxsize=None)
def _runs_for(L):
    idx = np.arange(L, dtype=np.int64)
    for win in _intervals_for(L):
        src = idx[win[0] - 1] if win[0] else idx[0]
        idx[win] = src
    masked = np.flatnonzero(idx != np.arange(L))
    runs = []
    if masked.size:
        start = prev = int(masked[0])
        val = int(idx[start])
        for i in masked[1:]:
            i = int(i)
            if i == prev + 1 and int(idx[i]) == val:
                prev = i
            else:
                runs.append((start, prev + 1, val))
                start = prev = i
                val = int(idx[i])
        runs.append((start, prev + 1, val))
    return tuple(runs)


def _mask_body(block_fills, fills_ref, x_ref, o_ref):
    pid = pl.program_id(0)
    o_ref[...] = x_ref[...]
    for b, fills in block_fills.items():
        @pl.when(pid == b)
        def _fill(fills=fills):
            for ls, le, r in fills:
                o_ref[ls:le] = jnp.broadcast_to(fills_ref[r], (le - ls,))


def kernel(x):
    L = x.shape[-1]
    runs = _runs_for(L)
    grid = pl.cdiv(L, _BLOCK)
    block_fills = {}
    for r, (s, e, _) in enumerate(runs):
        for b in range(s // _BLOCK, (e - 1) // _BLOCK + 1):
            lo, hi = max(s, b * _BLOCK), min(e, (b + 1) * _BLOCK)
            block_fills.setdefault(b, []).append((lo - b * _BLOCK, hi - b * _BLOCK, r))
    srcs = jnp.asarray([src for (_, _, src) in runs], dtype=jnp.int32)
    nf = max(len(runs), 1)
    fills = x[srcs] if len(runs) else jnp.zeros((1,), x.dtype)
    out = pl.pallas_call(
        functools.partial(_mask_body, block_fills),
        grid=(grid,),
        in_specs=[
            pl.BlockSpec((nf,), lambda i: (0,)),
            pl.BlockSpec((_BLOCK,), lambda i: (i,)),
        ],
        out_specs=pl.BlockSpec((_BLOCK,), lambda i: (i,)),
        out_shape=jax.ShapeDtypeStruct((L,), x.dtype),
        compiler_params=pltpu.CompilerParams(
            dimension_semantics=("parallel",),
        ),
    )(fills, x)
    return out
